# baseline (speedup 1.0000x reference)
from math import log

import jax
import jax.numpy as jnp
from jax import lax
from jax.experimental import pallas as pl
from jax.experimental.pallas import tpu as pltpu
from jax.experimental.pallas import tpu_sc as plsc

N = 10000
E = 320000
IN_C = 128
HID = 256
OUT_C = 64
L = 8
ALPHA = 0.1
THETA = 0.5

NB = 5
BN_ROWS = N // NB
HALF = HID // 2

NTEC = 16
K = 128
NCHUNK = 160
EPT = NCHUNK * K
NPAD = 10240
NACC = 10112
SLAB = 32
NSLAB = NCHUNK // SLAB
DW = 128

_f32 = jnp.float32


def _zero_rows(buf, rows, cols):
    def body(r, _):
        for j in range(cols // 16):
            buf[r, pl.ds(16 * j, 16)] = jnp.zeros((16,), _f32)
        return 0
    lax.fori_loop(0, rows, body, 0)


def _deg_body(colw_hbm, degs_hbm, colv, buf, acc2):
    c = lax.axis_index("c")
    s = lax.axis_index("s")
    pltpu.sync_copy(colw_hbm.at[c * NTEC + s], colv)

    rows_per_tec = NPAD // NTEC

    _zero_rows(buf, K, DW)

    def zc(k, _):
        pltpu.sync_copy(buf, acc2.at[pl.ds(s * rows_per_tec + K * k, K)])
        return 0
    lax.fori_loop(0, rows_per_tec // K, zc, 0)

    def fill(r, _):
        for q in range(DW // 16):
            buf[r, pl.ds(16 * q, 16)] = jnp.ones((16,), _f32)
        return 0
    lax.fori_loop(0, K, fill, 0)
    plsc.subcore_barrier()

    def sbody(j, _):
        pltpu.sync_copy(buf, acc2.at[colv.at[c * (NCHUNK // 2) + j]],
                        add=True)
        return 0
    lax.fori_loop(0, NCHUNK // 2, sbody, 0)
    plsc.subcore_barrier()

    def dr(k, _):
        off = s * rows_per_tec + K * k
        pltpu.sync_copy(acc2.at[pl.ds(off, K)],
                        degs_hbm.at[pl.ds(c * NPAD + off, K)])
        return 0
    lax.fori_loop(0, rows_per_tec // K, dr, 0)


def _sc_degree(colw):
    mesh = plsc.VectorSubcoreMesh(core_axis_name="c", subcore_axis_name="s")
    return pl.kernel(
        _deg_body,
        out_type=jax.ShapeDtypeStruct((2 * NPAD, DW), _f32),
        mesh=mesh,
        scratch_types=[
            pltpu.VMEM((NCHUNK, K), jnp.int32),
            pltpu.VMEM((K, DW), _f32),
            pltpu.VMEM_SHARED((NPAD, DW), _f32),
        ],
    )(colw)


def _spmm_body(hst_hbm, roww_hbm, colw_hbm, s2_hbm,
               ibr, ibc, g0, acc, semg0, semg1):
    c = lax.axis_index("c")
    s = lax.axis_index("s")
    w = c * NTEC + s

    def zrow(r, _):
        for j in range(K // 16):
            g0[0, r, pl.ds(16 * j, 16)] = jnp.zeros((16,), _f32)
        return 0
    lax.fori_loop(0, K, zrow, 0)
    rows_per_tec = NACC // NTEC

    def zc(k, _):
        pltpu.sync_copy(g0.at[0], acc.at[pl.ds(s * rows_per_tec + K * k, K)])
        return 0
    lax.fori_loop(0, 4, zc, 0)
    pltpu.sync_copy(g0.at[0, pl.ds(0, 120)],
                    acc.at[pl.ds(s * rows_per_tec + 4 * K, 120)])
    plsc.subcore_barrier()

    def slab_body(t, _):
        pltpu.sync_copy(roww_hbm.at[w, pl.ds(t * SLAB, SLAB)], ibr)
        pltpu.sync_copy(colw_hbm.at[w, pl.ds(t * SLAB, SLAB)], ibc)
        pltpu.make_async_copy(hst_hbm.at[ibr.at[0]], g0.at[0], semg0).start()

        def body(i, _):
            j0 = 2 * i
            j1 = 2 * i + 1
            pltpu.make_async_copy(hst_hbm.at[ibr.at[j0]], g0.at[0],
                                  semg0).wait()
            pltpu.make_async_copy(hst_hbm.at[ibr.at[j1]], g0.at[1],
                                  semg1).start()
            pltpu.sync_copy(g0.at[0], acc.at[ibc.at[j0]], add=True)
            pltpu.make_async_copy(hst_hbm.at[ibr.at[j1]], g0.at[1],
                                  semg1).wait()
            j2 = jnp.minimum(j1 + 1, SLAB - 1)
            pltpu.make_async_copy(hst_hbm.at[ibr.at[j2]], g0.at[0],
                                  semg0).start()
            pltpu.sync_copy(g0.at[1], acc.at[ibc.at[j1]], add=True)
            return 0
        lax.fori_loop(0, SLAB // 2, body, 0)
        pltpu.make_async_copy(hst_hbm.at[ibr.at[SLAB - 1]], g0.at[0],
                              semg0).wait()
        return 0
    lax.fori_loop(0, NSLAB, slab_body, 0)
    plsc.subcore_barrier()

    base = s * 632

    @pl.when(s < NTEC - 1)
    def _():
        pltpu.sync_copy(acc.at[pl.ds(base, 632)],
                        s2_hbm.at[pl.ds(c * N + base, 632)])

    @pl.when(s == NTEC - 1)
    def _():
        pltpu.sync_copy(acc.at[pl.ds(15 * 632, 520)],
                        s2_hbm.at[pl.ds(c * N + 15 * 632, 520)])


def _sc_spmm(hst, roww, colw):
    mesh = plsc.VectorSubcoreMesh(core_axis_name="c", subcore_axis_name="s")
    return pl.kernel(
        _spmm_body,
        out_type=jax.ShapeDtypeStruct((2 * N, HALF), _f32),
        mesh=mesh,
        scratch_types=[
            pltpu.VMEM((SLAB, K), jnp.int32),
            pltpu.VMEM((SLAB, K), jnp.int32),
            pltpu.VMEM((2, K, HALF), _f32),
            pltpu.VMEM_SHARED((NACC, HALF), _f32),
            pltpu.SemaphoreType.DMA,
            pltpu.SemaphoreType.DMA,
        ],
    )(hst, roww, colw)


def _init_body(x_ref, w_ref, b_ref, deg0_ref, deg1_ref, h_ref, hsa_ref,
               hsb_ref, dinv_ref):
    h = jnp.maximum(
        jnp.dot(x_ref[...], w_ref[...], preferred_element_type=jnp.float32)
        + b_ref[...], 0.0)
    deg = deg0_ref[...] + deg1_ref[...] + 1.0
    dinv = jax.lax.rsqrt(jnp.maximum(deg, 1e-12))
    hs = h * dinv
    h_ref[...] = h
    hsa_ref[...] = hs[:, :HALF]
    hsb_ref[...] = hs[:, HALF:]
    dinv_ref[...] = dinv


def _tc_init(x, lin0_W, lin0_b, deg0, deg1):
    blk = lambda c: pl.BlockSpec((BN_ROWS, c), lambda i: (i, 0))
    full = lambda r, c: pl.BlockSpec((r, c), lambda i: (0, 0))
    return pl.pallas_call(
        _init_body,
        grid=(NB,),
        in_specs=[blk(IN_C), full(IN_C, HID), full(1, HID), blk(1), blk(1)],
        out_specs=(blk(HID), blk(HALF), blk(HALF), blk(1)),
        out_shape=(
            jax.ShapeDtypeStruct((N, HID), _f32),
            jax.ShapeDtypeStruct((N, HALF), _f32),
            jax.ShapeDtypeStruct((N, HALF), _f32),
            jax.ShapeDtypeStruct((N, 1), _f32),
        ),
    )(x, lin0_W, lin0_b.reshape(1, HID), deg0.reshape(N, 1),
      deg1.reshape(N, 1))


def _make_mm_body(beta):
    def _mm_body(sa_ref, sb_ref, hsa_ref, hsb_ref, h0_ref, dinv_ref, w_ref,
                 t_ref, acc_ref):
        dinv = dinv_ref[...]
        s = jnp.concatenate([sa_ref[...], sb_ref[...]], axis=1)
        hs = jnp.concatenate([hsa_ref[...], hsb_ref[...]], axis=1)
        agg = (s + hs) * dinv
        z = (1.0 - ALPHA) * agg + ALPHA * h0_ref[...]
        t = (1.0 - beta) * z + beta * jnp.dot(
            z, w_ref[...], preferred_element_type=jnp.float32)
        t_ref[...] = t
        part = jnp.concatenate(
            [jnp.sum(t, axis=0, keepdims=True),
             jnp.sum(t * t, axis=0, keepdims=True)], axis=0)

        @pl.when(pl.program_id(0) == 0)
        def _():
            acc_ref[...] = part

        @pl.when(pl.program_id(0) != 0)
        def _():
            acc_ref[...] += part
    return _mm_body


def _bn_body(t_ref, acc_ref, g_ref, b_ref, dinv_ref, h_ref, hsa_ref,
             hsb_ref):
    mu = acc_ref[0:1, :] * (1.0 / N)
    var = acc_ref[1:2, :] * (1.0 / N) - mu * mu
    hn = jnp.maximum(
        (t_ref[...] - mu) * jax.lax.rsqrt(var + 1e-5) * g_ref[...]
        + b_ref[...], 0.0)
    hs = hn * dinv_ref[...]
    h_ref[...] = hn
    hsa_ref[...] = hs[:, :HALF]
    hsb_ref[...] = hs[:, HALF:]


def _tc_layer(beta, s2, hst, h0, dinv, W, g, b):
    blk = lambda c: pl.BlockSpec((BN_ROWS, c), lambda i: (i, 0))
    blk_hi = lambda c: pl.BlockSpec((BN_ROWS, c), lambda i: (i + NB, 0))
    full = lambda r, c: pl.BlockSpec((r, c), lambda i: (0, 0))
    t, acc = pl.pallas_call(
        _make_mm_body(beta),
        grid=(NB,),
        in_specs=[blk(HALF), blk_hi(HALF), blk(HALF), blk_hi(HALF),
                  blk(HID), blk(1), full(HID, HID)],
        out_specs=(blk(HID), full(2, HID)),
        out_shape=(
            jax.ShapeDtypeStruct((N, HID), _f32),
            jax.ShapeDtypeStruct((2, HID), _f32),
        ),
    )(s2, s2, hst, hst, h0, dinv, W)
    return pl.pallas_call(
        _bn_body,
        grid=(NB,),
        in_specs=[blk(HID), full(2, HID), full(1, HID), full(1, HID), blk(1)],
        out_specs=(blk(HID), blk(HALF), blk(HALF)),
        out_shape=(
            jax.ShapeDtypeStruct((N, HID), _f32),
            jax.ShapeDtypeStruct((N, HALF), _f32),
            jax.ShapeDtypeStruct((N, HALF), _f32),
        ),
    )(t, acc, g.reshape(1, HID), b.reshape(1, HID), dinv)


def _final_body(h_ref, w_ref, b_ref, out_ref):
    out_ref[...] = jnp.dot(
        h_ref[...], w_ref[...], preferred_element_type=jnp.float32
    ) + b_ref[...]


def _tc_final(h, lin1_W, lin1_b):
    blk = lambda c: pl.BlockSpec((BN_ROWS, c), lambda i: (i, 0))
    full = lambda r, c: pl.BlockSpec((r, c), lambda i: (0, 0))
    return pl.pallas_call(
        _final_body,
        grid=(NB,),
        in_specs=[blk(HID), full(HID, OUT_C), full(1, OUT_C)],
        out_specs=blk(OUT_C),
        out_shape=jax.ShapeDtypeStruct((N, OUT_C), _f32),
    )(h, lin1_W, lin1_b.reshape(1, OUT_C))


def kernel(x, edge_index, lin0_W, lin0_b, conv_W, bn_gamma, bn_beta,
           lin1_W, lin1_b):
    row = edge_index[0]
    col = edge_index[1]

    rowp = jnp.pad(row.reshape(NTEC, E // NTEC), ((0, 0), (0, EPT - E // NTEC)))
    colp = jnp.pad(col.reshape(NTEC, E // NTEC), ((0, 0), (0, EPT - E // NTEC)),
                   constant_values=N)
    roww = jnp.concatenate([rowp, rowp + N], axis=0).reshape(2 * NTEC, NCHUNK, K)
    colw = jnp.concatenate([colp, colp], axis=0).reshape(2 * NTEC, NCHUNK, K)

    degs = _sc_degree(colw)

    h0, hsa, hsb, dinv = _tc_init(x, lin0_W, lin0_b,
                                  degs[:N, 0], degs[NPAD:NPAD + N, 0])

    h = h0
    hst = jnp.concatenate([hsa, hsb], axis=0)
    for l in range(L):
        s2 = _sc_spmm(hst, roww, colw)
        beta = log(THETA / (l + 1) + 1.0)
        h, hsa, hsb = _tc_layer(beta, s2, hst, h0, dinv,
                                conv_W[l], bn_gamma[l], bn_beta[l])
        hst = jnp.concatenate([hsa, hsb], axis=0)

    return _tc_final(h, lin1_W, lin1_b)


# speedup vs baseline: 1.0022x; 1.0022x over previous
from math import log

import jax
import jax.numpy as jnp
from jax import lax
from jax.experimental import pallas as pl
from jax.experimental.pallas import tpu as pltpu
from jax.experimental.pallas import tpu_sc as plsc

N = 10000
E = 320000
IN_C = 128
HID = 256
OUT_C = 64
L = 8
ALPHA = 0.1
THETA = 0.5

NB = 5
BN_ROWS = N // NB
HALF = HID // 2

NTEC = 16
K = 128
NCHUNK = 160
EPT = NCHUNK * K
NPAD = 10240
NACC = 10112
SLAB = 16
NSLAB = NCHUNK // SLAB
DW = 128

_f32 = jnp.float32


def _zero_rows(buf, rows, cols):
    def body(r, _):
        for j in range(cols // 16):
            buf[r, pl.ds(16 * j, 16)] = jnp.zeros((16,), _f32)
        return 0
    lax.fori_loop(0, rows, body, 0)


def _deg_body(colw_hbm, degs_hbm, colv, buf, acc2):
    c = lax.axis_index("c")
    s = lax.axis_index("s")
    pltpu.sync_copy(colw_hbm.at[c * NTEC + s], colv)

    rows_per_tec = NPAD // NTEC

    _zero_rows(buf, K, DW)

    def zc(k, _):
        pltpu.sync_copy(buf, acc2.at[pl.ds(s * rows_per_tec + K * k, K)])
        return 0
    lax.fori_loop(0, rows_per_tec // K, zc, 0)

    def fill(r, _):
        for q in range(DW // 16):
            buf[r, pl.ds(16 * q, 16)] = jnp.ones((16,), _f32)
        return 0
    lax.fori_loop(0, K, fill, 0)
    plsc.subcore_barrier()

    def sbody(j, _):
        pltpu.sync_copy(buf, acc2.at[colv.at[c * (NCHUNK // 2) + j]],
                        add=True)
        return 0
    lax.fori_loop(0, NCHUNK // 2, sbody, 0)
    plsc.subcore_barrier()

    def dr(k, _):
        off = s * rows_per_tec + K * k
        pltpu.sync_copy(acc2.at[pl.ds(off, K)],
                        degs_hbm.at[pl.ds(c * NPAD + off, K)])
        return 0
    lax.fori_loop(0, rows_per_tec // K, dr, 0)


def _sc_degree(colw):
    mesh = plsc.VectorSubcoreMesh(core_axis_name="c", subcore_axis_name="s")
    return pl.kernel(
        _deg_body,
        out_type=jax.ShapeDtypeStruct((2 * NPAD, DW), _f32),
        mesh=mesh,
        scratch_types=[
            pltpu.VMEM((NCHUNK, K), jnp.int32),
            pltpu.VMEM((K, DW), _f32),
            pltpu.VMEM_SHARED((NPAD, DW), _f32),
        ],
    )(colw)


def _spmm_body(hst_hbm, roww_hbm, colw_hbm, s2_hbm,
               ibr, ibc, g0, acc, semg0, semg1, semi0, semi1):
    c = lax.axis_index("c")
    s = lax.axis_index("s")
    w = c * NTEC + s

    def zrow(r, _):
        for j in range(K // 16):
            g0[0, r, pl.ds(16 * j, 16)] = jnp.zeros((16,), _f32)
        return 0
    lax.fori_loop(0, K, zrow, 0)
    rows_per_tec = NACC // NTEC

    def zc(k, _):
        pltpu.sync_copy(g0.at[0], acc.at[pl.ds(s * rows_per_tec + K * k, K)])
        return 0
    lax.fori_loop(0, 4, zc, 0)
    pltpu.sync_copy(g0.at[0, pl.ds(0, 120)],
                    acc.at[pl.ds(s * rows_per_tec + 4 * K, 120)])
    plsc.subcore_barrier()

    def idx_load(t, p):
        pltpu.make_async_copy(roww_hbm.at[w, pl.ds(t * SLAB, SLAB)],
                              ibr.at[p], semi0).start()
        pltpu.make_async_copy(colw_hbm.at[w, pl.ds(t * SLAB, SLAB)],
                              ibc.at[p], semi1).start()

    def idx_wait(t, p):
        pltpu.make_async_copy(roww_hbm.at[w, pl.ds(t * SLAB, SLAB)],
                              ibr.at[p], semi0).wait()
        pltpu.make_async_copy(colw_hbm.at[w, pl.ds(t * SLAB, SLAB)],
                              ibc.at[p], semi1).wait()

    idx_load(0, 0)

    def slab_body(t, _):
        p = lax.rem(t, 2)
        idx_wait(t, p)
        tn = jnp.minimum(t + 1, NSLAB - 1)
        idx_load(tn, 1 - p)
        ibrp = ibr.at[p]
        ibcp = ibc.at[p]
        pltpu.make_async_copy(hst_hbm.at[ibrp.at[0]], g0.at[0], semg0).start()

        def body(i, _):
            j0 = 2 * i
            j1 = 2 * i + 1
            pltpu.make_async_copy(hst_hbm.at[ibrp.at[j0]], g0.at[0],
                                  semg0).wait()
            pltpu.make_async_copy(hst_hbm.at[ibrp.at[j1]], g0.at[1],
                                  semg1).start()
            pltpu.sync_copy(g0.at[0], acc.at[ibcp.at[j0]], add=True)
            pltpu.make_async_copy(hst_hbm.at[ibrp.at[j1]], g0.at[1],
                                  semg1).wait()
            j2 = jnp.minimum(j1 + 1, SLAB - 1)
            pltpu.make_async_copy(hst_hbm.at[ibrp.at[j2]], g0.at[0],
                                  semg0).start()
            pltpu.sync_copy(g0.at[1], acc.at[ibcp.at[j1]], add=True)
            return 0
        lax.fori_loop(0, SLAB // 2, body, 0)
        pltpu.make_async_copy(hst_hbm.at[ibrp.at[SLAB - 1]], g0.at[0],
                              semg0).wait()
        return 0
    lax.fori_loop(0, NSLAB, slab_body, 0)
    idx_wait(NSLAB - 1, lax.rem(NSLAB, 2))
    plsc.subcore_barrier()

    base = s * 632

    @pl.when(s < NTEC - 1)
    def _():
        pltpu.sync_copy(acc.at[pl.ds(base, 632)],
                        s2_hbm.at[pl.ds(c * N + base, 632)])

    @pl.when(s == NTEC - 1)
    def _():
        pltpu.sync_copy(acc.at[pl.ds(15 * 632, 520)],
                        s2_hbm.at[pl.ds(c * N + 15 * 632, 520)])


def _sc_spmm(hst, roww, colw):
    mesh = plsc.VectorSubcoreMesh(core_axis_name="c", subcore_axis_name="s")
    return pl.kernel(
        _spmm_body,
        out_type=jax.ShapeDtypeStruct((2 * N, HALF), _f32),
        mesh=mesh,
        scratch_types=[
            pltpu.VMEM((2, SLAB, K), jnp.int32),
            pltpu.VMEM((2, SLAB, K), jnp.int32),
            pltpu.VMEM((2, K, HALF), _f32),
            pltpu.VMEM_SHARED((NACC, HALF), _f32),
            pltpu.SemaphoreType.DMA,
            pltpu.SemaphoreType.DMA,
            pltpu.SemaphoreType.DMA,
            pltpu.SemaphoreType.DMA,
        ],
    )(hst, roww, colw)


def _init_body(x_ref, w_ref, b_ref, deg0_ref, deg1_ref, h_ref, hsa_ref,
               hsb_ref, dinv_ref):
    h = jnp.maximum(
        jnp.dot(x_ref[...], w_ref[...], preferred_element_type=jnp.float32)
        + b_ref[...], 0.0)
    deg = deg0_ref[...] + deg1_ref[...] + 1.0
    dinv = jax.lax.rsqrt(jnp.maximum(deg, 1e-12))
    hs = h * dinv
    h_ref[...] = h
    hsa_ref[...] = hs[:, :HALF]
    hsb_ref[...] = hs[:, HALF:]
    dinv_ref[...] = dinv


def _tc_init(x, lin0_W, lin0_b, deg0, deg1):
    blk = lambda c: pl.BlockSpec((BN_ROWS, c), lambda i: (i, 0))
    full = lambda r, c: pl.BlockSpec((r, c), lambda i: (0, 0))
    return pl.pallas_call(
        _init_body,
        grid=(NB,),
        in_specs=[blk(IN_C), full(IN_C, HID), full(1, HID), blk(1), blk(1)],
        out_specs=(blk(HID), blk(HALF), blk(HALF), blk(1)),
        out_shape=(
            jax.ShapeDtypeStruct((N, HID), _f32),
            jax.ShapeDtypeStruct((N, HALF), _f32),
            jax.ShapeDtypeStruct((N, HALF), _f32),
            jax.ShapeDtypeStruct((N, 1), _f32),
        ),
    )(x, lin0_W, lin0_b.reshape(1, HID), deg0.reshape(N, 1),
      deg1.reshape(N, 1))


def _make_mm_body(beta):
    def _mm_body(sa_ref, sb_ref, hsa_ref, hsb_ref, h0_ref, dinv_ref, w_ref,
                 t_ref, acc_ref):
        dinv = dinv_ref[...]
        s = jnp.concatenate([sa_ref[...], sb_ref[...]], axis=1)
        hs = jnp.concatenate([hsa_ref[...], hsb_ref[...]], axis=1)
        agg = (s + hs) * dinv
        z = (1.0 - ALPHA) * agg + ALPHA * h0_ref[...]
        t = (1.0 - beta) * z + beta * jnp.dot(
            z, w_ref[...], preferred_element_type=jnp.float32)
        t_ref[...] = t
        part = jnp.concatenate(
            [jnp.sum(t, axis=0, keepdims=True),
             jnp.sum(t * t, axis=0, keepdims=True)], axis=0)

        @pl.when(pl.program_id(0) == 0)
        def _():
            acc_ref[...] = part

        @pl.when(pl.program_id(0) != 0)
        def _():
            acc_ref[...] += part
    return _mm_body


def _bn_body(t_ref, acc_ref, g_ref, b_ref, dinv_ref, h_ref, hsa_ref,
             hsb_ref):
    mu = acc_ref[0:1, :] * (1.0 / N)
    var = acc_ref[1:2, :] * (1.0 / N) - mu * mu
    hn = jnp.maximum(
        (t_ref[...] - mu) * jax.lax.rsqrt(var + 1e-5) * g_ref[...]
        + b_ref[...], 0.0)
    hs = hn * dinv_ref[...]
    h_ref[...] = hn
    hsa_ref[...] = hs[:, :HALF]
    hsb_ref[...] = hs[:, HALF:]


def _tc_layer(beta, s2, hst, h0, dinv, W, g, b):
    blk = lambda c: pl.BlockSpec((BN_ROWS, c), lambda i: (i, 0))
    blk_hi = lambda c: pl.BlockSpec((BN_ROWS, c), lambda i: (i + NB, 0))
    full = lambda r, c: pl.BlockSpec((r, c), lambda i: (0, 0))
    t, acc = pl.pallas_call(
        _make_mm_body(beta),
        grid=(NB,),
        in_specs=[blk(HALF), blk_hi(HALF), blk(HALF), blk_hi(HALF),
                  blk(HID), blk(1), full(HID, HID)],
        out_specs=(blk(HID), full(2, HID)),
        out_shape=(
            jax.ShapeDtypeStruct((N, HID), _f32),
            jax.ShapeDtypeStruct((2, HID), _f32),
        ),
    )(s2, s2, hst, hst, h0, dinv, W)
    return pl.pallas_call(
        _bn_body,
        grid=(NB,),
        in_specs=[blk(HID), full(2, HID), full(1, HID), full(1, HID), blk(1)],
        out_specs=(blk(HID), blk(HALF), blk(HALF)),
        out_shape=(
            jax.ShapeDtypeStruct((N, HID), _f32),
            jax.ShapeDtypeStruct((N, HALF), _f32),
            jax.ShapeDtypeStruct((N, HALF), _f32),
        ),
    )(t, acc, g.reshape(1, HID), b.reshape(1, HID), dinv)


def _final_body(h_ref, w_ref, b_ref, out_ref):
    out_ref[...] = jnp.dot(
        h_ref[...], w_ref[...], preferred_element_type=jnp.float32
    ) + b_ref[...]


def _tc_final(h, lin1_W, lin1_b):
    blk = lambda c: pl.BlockSpec((BN_ROWS, c), lambda i: (i, 0))
    full = lambda r, c: pl.BlockSpec((r, c), lambda i: (0, 0))
    return pl.pallas_call(
        _final_body,
        grid=(NB,),
        in_specs=[blk(HID), full(HID, OUT_C), full(1, OUT_C)],
        out_specs=blk(OUT_C),
        out_shape=jax.ShapeDtypeStruct((N, OUT_C), _f32),
    )(h, lin1_W, lin1_b.reshape(1, OUT_C))


def kernel(x, edge_index, lin0_W, lin0_b, conv_W, bn_gamma, bn_beta,
           lin1_W, lin1_b):
    row = edge_index[0]
    col = edge_index[1]

    rowp = jnp.pad(row.reshape(NTEC, E // NTEC), ((0, 0), (0, EPT - E // NTEC)))
    colp = jnp.pad(col.reshape(NTEC, E // NTEC), ((0, 0), (0, EPT - E // NTEC)),
                   constant_values=N)
    roww = jnp.concatenate([rowp, rowp + N], axis=0).reshape(2 * NTEC, NCHUNK, K)
    colw = jnp.concatenate([colp, colp], axis=0).reshape(2 * NTEC, NCHUNK, K)

    degs = _sc_degree(colw)

    h0, hsa, hsb, dinv = _tc_init(x, lin0_W, lin0_b,
                                  degs[:N, 0], degs[NPAD:NPAD + N, 0])

    h = h0
    hst = jnp.concatenate([hsa, hsb], axis=0)
    for l in range(L):
        s2 = _sc_spmm(hst, roww, colw)
        beta = log(THETA / (l + 1) + 1.0)
        h, hsa, hsb = _tc_layer(beta, s2, hst, h0, dinv,
                                conv_W[l], bn_gamma[l], bn_beta[l])
        hst = jnp.concatenate([hsa, hsb], axis=0)

    return _tc_final(h, lin1_W, lin1_b)


# speedup vs baseline: 1.0034x; 1.0012x over previous
from math import log

import jax
import jax.numpy as jnp
from jax import lax
from jax.experimental import pallas as pl
from jax.experimental.pallas import tpu as pltpu
from jax.experimental.pallas import tpu_sc as plsc

N = 10000
E = 320000
IN_C = 128
HID = 256
OUT_C = 64
L = 8
ALPHA = 0.1
THETA = 0.5

NB = 5
BN_ROWS = N // NB
HALF = HID // 2

NTEC = 16
K = 128
NCHUNK = 160
EPT = NCHUNK * K
NPAD = 10240
NACC = 10112
SLAB = 32
NSLAB = NCHUNK // SLAB
DW = 128

_f32 = jnp.float32


def _zero_rows(buf, rows, cols):
    def body(r, _):
        for j in range(cols // 16):
            buf[r, pl.ds(16 * j, 16)] = jnp.zeros((16,), _f32)
        return 0
    lax.fori_loop(0, rows, body, 0)


def _deg_body(colw_hbm, degs_hbm, colv, buf, acc2):
    c = lax.axis_index("c")
    s = lax.axis_index("s")
    pltpu.sync_copy(colw_hbm.at[c * NTEC + s], colv)

    rows_per_tec = NPAD // NTEC

    _zero_rows(buf, K, DW)

    def zc(k, _):
        pltpu.sync_copy(buf, acc2.at[pl.ds(s * rows_per_tec + K * k, K)])
        return 0
    lax.fori_loop(0, rows_per_tec // K, zc, 0)

    def fill(r, _):
        for q in range(DW // 16):
            buf[r, pl.ds(16 * q, 16)] = jnp.ones((16,), _f32)
        return 0
    lax.fori_loop(0, K, fill, 0)
    plsc.subcore_barrier()

    def sbody(j, _):
        pltpu.sync_copy(buf, acc2.at[colv.at[c * (NCHUNK // 2) + j]],
                        add=True)
        return 0
    lax.fori_loop(0, NCHUNK // 2, sbody, 0)
    plsc.subcore_barrier()

    def dr(k, _):
        off = s * rows_per_tec + K * k
        pltpu.sync_copy(acc2.at[pl.ds(off, K)],
                        degs_hbm.at[pl.ds(c * NPAD + off, K)])
        return 0
    lax.fori_loop(0, rows_per_tec // K, dr, 0)


def _sc_degree(colw):
    mesh = plsc.VectorSubcoreMesh(core_axis_name="c", subcore_axis_name="s")
    return pl.kernel(
        _deg_body,
        out_type=jax.ShapeDtypeStruct((2 * NPAD, DW), _f32),
        mesh=mesh,
        scratch_types=[
            pltpu.VMEM((NCHUNK, K), jnp.int32),
            pltpu.VMEM((K, DW), _f32),
            pltpu.VMEM_SHARED((NPAD, DW), _f32),
        ],
    )(colw)


def _spmm_body(hst_hbm, roww_hbm, colw_hbm, s2_hbm,
               ibr, ibc, g0, acc, semg0, semg1):
    c = lax.axis_index("c")
    s = lax.axis_index("s")
    w = c * NTEC + s

    def zrow(r, _):
        for j in range(K // 16):
            g0[0, r, pl.ds(16 * j, 16)] = jnp.zeros((16,), _f32)
        return 0
    lax.fori_loop(0, K, zrow, 0)
    rows_per_tec = NACC // NTEC

    def zc(k, _):
        pltpu.sync_copy(g0.at[0], acc.at[pl.ds(s * rows_per_tec + K * k, K)])
        return 0
    lax.fori_loop(0, 4, zc, 0)
    pltpu.sync_copy(g0.at[0, pl.ds(0, 120)],
                    acc.at[pl.ds(s * rows_per_tec + 4 * K, 120)])
    plsc.subcore_barrier()

    def slab_body(t, _):
        pltpu.sync_copy(roww_hbm.at[w, pl.ds(t * SLAB, SLAB)], ibr)
        pltpu.sync_copy(colw_hbm.at[w, pl.ds(t * SLAB, SLAB)], ibc)
        pltpu.make_async_copy(hst_hbm.at[ibr.at[0]], g0.at[0], semg0).start()

        def body(i, _):
            j0 = 2 * i
            j1 = 2 * i + 1
            pltpu.make_async_copy(hst_hbm.at[ibr.at[j0]], g0.at[0],
                                  semg0).wait()
            pltpu.make_async_copy(hst_hbm.at[ibr.at[j1]], g0.at[1],
                                  semg1).start()
            pltpu.sync_copy(g0.at[0], acc.at[ibc.at[j0]], add=True)
            pltpu.make_async_copy(hst_hbm.at[ibr.at[j1]], g0.at[1],
                                  semg1).wait()
            j2 = jnp.minimum(j1 + 1, SLAB - 1)
            pltpu.make_async_copy(hst_hbm.at[ibr.at[j2]], g0.at[0],
                                  semg0).start()
            pltpu.sync_copy(g0.at[1], acc.at[ibc.at[j1]], add=True)
            return 0
        lax.fori_loop(0, SLAB // 2, body, 0)
        pltpu.make_async_copy(hst_hbm.at[ibr.at[SLAB - 1]], g0.at[0],
                              semg0).wait()
        return 0
    lax.fori_loop(0, NSLAB, slab_body, 0)
    plsc.subcore_barrier()

    base = s * 632

    @pl.when(s < NTEC - 1)
    def _():
        pltpu.sync_copy(acc.at[pl.ds(base, 632)],
                        s2_hbm.at[pl.ds(c * N + base, 632)])

    @pl.when(s == NTEC - 1)
    def _():
        pltpu.sync_copy(acc.at[pl.ds(15 * 632, 520)],
                        s2_hbm.at[pl.ds(c * N + 15 * 632, 520)])


def _sc_spmm(hst, roww, colw):
    mesh = plsc.VectorSubcoreMesh(core_axis_name="c", subcore_axis_name="s")
    return pl.kernel(
        _spmm_body,
        out_type=jax.ShapeDtypeStruct((2 * N, HALF), _f32),
        mesh=mesh,
        scratch_types=[
            pltpu.VMEM((SLAB, K), jnp.int32),
            pltpu.VMEM((SLAB, K), jnp.int32),
            pltpu.VMEM((2, K, HALF), _f32),
            pltpu.VMEM_SHARED((NACC, HALF), _f32),
            pltpu.SemaphoreType.DMA,
            pltpu.SemaphoreType.DMA,
        ],
    )(hst, roww, colw)


def _init_body(x_ref, w_ref, b_ref, deg0_ref, deg1_ref, h_ref, hsa_ref,
               hsb_ref, dinv_ref):
    h = jnp.maximum(
        jnp.dot(x_ref[...], w_ref[...], preferred_element_type=jnp.float32)
        + b_ref[...], 0.0)
    deg = deg0_ref[...] + deg1_ref[...] + 1.0
    dinv = jax.lax.rsqrt(jnp.maximum(deg, 1e-12))
    hs = h * dinv
    h_ref[...] = h
    hsa_ref[...] = hs[:, :HALF]
    hsb_ref[...] = hs[:, HALF:]
    dinv_ref[...] = dinv


def _tc_init(x, lin0_W, lin0_b, deg0, deg1):
    blk = lambda c: pl.BlockSpec((BN_ROWS, c), lambda i: (i, 0))
    full = lambda r, c: pl.BlockSpec((r, c), lambda i: (0, 0))
    return pl.pallas_call(
        _init_body,
        grid=(NB,),
        in_specs=[blk(IN_C), full(IN_C, HID), full(1, HID), blk(1), blk(1)],
        out_specs=(blk(HID), blk(HALF), blk(HALF), blk(1)),
        out_shape=(
            jax.ShapeDtypeStruct((N, HID), _f32),
            jax.ShapeDtypeStruct((N, HALF), _f32),
            jax.ShapeDtypeStruct((N, HALF), _f32),
            jax.ShapeDtypeStruct((N, 1), _f32),
        ),
    )(x, lin0_W, lin0_b.reshape(1, HID), deg0.reshape(N, 1),
      deg1.reshape(N, 1))


def _make_mm_body(beta):
    def _mm_body(sa_ref, sb_ref, hsa_ref, hsb_ref, h0_ref, dinv_ref, w_ref,
                 t_ref, acc_ref):
        dinv = dinv_ref[...]
        s = jnp.concatenate([sa_ref[...], sb_ref[...]], axis=1)
        hs = jnp.concatenate([hsa_ref[...], hsb_ref[...]], axis=1)
        agg = (s + hs) * dinv
        z = (1.0 - ALPHA) * agg + ALPHA * h0_ref[...]
        t = (1.0 - beta) * z + beta * jnp.dot(
            z, w_ref[...], preferred_element_type=jnp.float32)
        t_ref[...] = t
        part = jnp.concatenate(
            [jnp.sum(t, axis=0, keepdims=True),
             jnp.sum(t * t, axis=0, keepdims=True)], axis=0)

        @pl.when(pl.program_id(0) == 0)
        def _():
            acc_ref[...] = part

        @pl.when(pl.program_id(0) != 0)
        def _():
            acc_ref[...] += part
    return _mm_body


def _bn_body(t_ref, acc_ref, g_ref, b_ref, dinv_ref, h_ref, hsa_ref,
             hsb_ref):
    mu = acc_ref[0:1, :] * (1.0 / N)
    var = acc_ref[1:2, :] * (1.0 / N) - mu * mu
    hn = jnp.maximum(
        (t_ref[...] - mu) * jax.lax.rsqrt(var + 1e-5) * g_ref[...]
        + b_ref[...], 0.0)
    hs = hn * dinv_ref[...]
    h_ref[...] = hn
    hsa_ref[...] = hs[:, :HALF]
    hsb_ref[...] = hs[:, HALF:]


def _tc_layer(beta, s2, hst, h0, dinv, W, g, b):
    blk = lambda c: pl.BlockSpec((BN_ROWS, c), lambda i: (i, 0))
    blk_hi = lambda c: pl.BlockSpec((BN_ROWS, c), lambda i: (i + NB, 0))
    full = lambda r, c: pl.BlockSpec((r, c), lambda i: (0, 0))
    t, acc = pl.pallas_call(
        _make_mm_body(beta),
        grid=(NB,),
        in_specs=[blk(HALF), blk_hi(HALF), blk(HALF), blk_hi(HALF),
                  blk(HID), blk(1), full(HID, HID)],
        out_specs=(blk(HID), full(2, HID)),
        out_shape=(
            jax.ShapeDtypeStruct((N, HID), _f32),
            jax.ShapeDtypeStruct((2, HID), _f32),
        ),
    )(s2, s2, hst, hst, h0, dinv, W)
    return pl.pallas_call(
        _bn_body,
        grid=(NB,),
        in_specs=[blk(HID), full(2, HID), full(1, HID), full(1, HID), blk(1)],
        out_specs=(blk(HID), blk(HALF), blk(HALF)),
        out_shape=(
            jax.ShapeDtypeStruct((N, HID), _f32),
            jax.ShapeDtypeStruct((N, HALF), _f32),
            jax.ShapeDtypeStruct((N, HALF), _f32),
        ),
    )(t, acc, g.reshape(1, HID), b.reshape(1, HID), dinv)


def _final_body(h_ref, w_ref, b_ref, out_ref):
    out_ref[...] = jnp.dot(
        h_ref[...], w_ref[...], preferred_element_type=jnp.float32
    ) + b_ref[...]


def _tc_final(h, lin1_W, lin1_b):
    blk = lambda c: pl.BlockSpec((BN_ROWS, c), lambda i: (i, 0))
    full = lambda r, c: pl.BlockSpec((r, c), lambda i: (0, 0))
    return pl.pallas_call(
        _final_body,
        grid=(NB,),
        in_specs=[blk(HID), full(HID, OUT_C), full(1, OUT_C)],
        out_specs=blk(OUT_C),
        out_shape=jax.ShapeDtypeStruct((N, OUT_C), _f32),
    )(h, lin1_W, lin1_b.reshape(1, OUT_C))


def kernel(x, edge_index, lin0_W, lin0_b, conv_W, bn_gamma, bn_beta,
           lin1_W, lin1_b):
    row = edge_index[0]
    col = edge_index[1]

    rowp = jnp.pad(row.reshape(NTEC, E // NTEC), ((0, 0), (0, EPT - E // NTEC)))
    colp = jnp.pad(col.reshape(NTEC, E // NTEC), ((0, 0), (0, EPT - E // NTEC)),
                   constant_values=N)
    roww = jnp.concatenate([rowp, rowp + N], axis=0).reshape(2 * NTEC, NCHUNK, K)
    colw = jnp.concatenate([colp, colp], axis=0).reshape(2 * NTEC, NCHUNK, K)

    degs = _sc_degree(colw)

    h0, hsa, hsb, dinv = _tc_init(x, lin0_W, lin0_b,
                                  degs[:N, 0], degs[NPAD:NPAD + N, 0])

    h = h0
    hst = jnp.concatenate([hsa, hsb], axis=0)
    for l in range(L):
        s2 = _sc_spmm(hst, roww, colw)
        beta = log(THETA / (l + 1) + 1.0)
        h, hsa, hsb = _tc_layer(beta, s2, hst, h0, dinv,
                                conv_W[l], bn_gamma[l], bn_beta[l])
        hst = jnp.concatenate([hsa, hsb], axis=0)

    return _tc_final(h, lin1_W, lin1_b)
